# baseline (speedup 1.0000x reference)
import functools
import jax
import jax.numpy as jnp
from jax import lax
from jax.experimental import pallas as pl
from jax.experimental.pallas import tpu as pltpu
from jax.experimental.pallas import tpu_sc as plsc

DMAX = 16
NB = 8
CW = 64
L = 16


def kernel(x, delays):
    B, T, C = x.shape
    Tp = T + DMAX
    CH = Tp // NB
    d32 = delays.astype(jnp.int32)
    mesh = plsc.VectorSubcoreMesh(core_axis_name="c", subcore_axis_name="s")

    @functools.partial(
        pl.kernel,
        out_type=jax.ShapeDtypeStruct((B, Tp, C), x.dtype),
        mesh=mesh,
        scratch_types=[
            pltpu.VMEM((CH + DMAX, CW), jnp.float32),
            pltpu.VMEM((CH, CW), jnp.float32),
            pltpu.VMEM((C,), jnp.int32),
            pltpu.SemaphoreType.DMA,
        ],
        compiler_params=pltpu.CompilerParams(
            use_tc_tiling_on_sc=False, needs_layout_passes=False),
    )
    def sc_kernel(x_hbm, d_hbm, o_hbm, zbuf, obuf, dbuf, sem):
        wid = lax.axis_index("s") * 2 + lax.axis_index("c")
        b = wid // NB
        tc = wid % NB
        t0 = tc * CH
        pltpu.sync_copy(d_hbm, dbuf)

        @pl.loop(0, C // CW)
        def _(ci):
            c0 = ci * CW

            @pl.when(tc == 0)
            def _():
                @pl.loop(0, DMAX)
                def _(r):
                    for g in range(CW // L):
                        zbuf[r, pl.ds(g * L, L)] = jnp.zeros((L,), jnp.float32)
                pltpu.async_copy(
                    x_hbm.at[b, pl.ds(0, CH), pl.ds(c0, CW)],
                    zbuf.at[pl.ds(DMAX, CH)], sem).wait()

            @pl.when(tc == NB - 1)
            def _():
                @pl.loop(CH, CH + DMAX)
                def _(r):
                    for g in range(CW // L):
                        zbuf[r, pl.ds(g * L, L)] = jnp.zeros((L,), jnp.float32)
                pltpu.async_copy(
                    x_hbm.at[b, pl.ds(T - CH, CH), pl.ds(c0, CW)],
                    zbuf.at[pl.ds(0, CH)], sem).wait()

            @pl.when(jnp.logical_and(tc > 0, tc < NB - 1))
            def _():
                pltpu.async_copy(
                    x_hbm.at[b, pl.ds(t0 - DMAX, CH + DMAX), pl.ds(c0, CW)],
                    zbuf, sem).wait()

            for g in range(CW // L):
                base = DMAX - dbuf[pl.ds(c0 + g * L, L)]
                col = lax.iota(jnp.int32, L) + g * L

                @pl.loop(0, CH)
                def _(i, base=base, col=col, g=g):
                    v = plsc.load_gather(zbuf, [base + i, col])
                    obuf[i, pl.ds(g * L, L)] = v

            pltpu.async_copy(
                obuf, o_hbm.at[b, pl.ds(t0, CH), pl.ds(c0, CW)], sem).wait()

    return sc_kernel(x, d32)


# speedup vs baseline: 6.7949x; 6.7949x over previous
import jax
import jax.numpy as jnp
from jax.experimental import pallas as pl
from jax.experimental.pallas import tpu as pltpu

DMAX = 16
CB = 256


def _shift_kernel(d_ref, x_ref, o_ref):
    x = x_ref[0]
    d = d_ref[...]
    z = jnp.pad(x, ((DMAX, DMAX), (0, 0)))
    w = z
    for k in range(5):
        s = 1 << k
        mask = ((d >> k) & 1) == 1
        shifted = jnp.roll(w, s, axis=0)
        w = jnp.where(mask, shifted, w)
    o_ref[0] = w[DMAX:]


def kernel(x, delays):
    B, T, C = x.shape
    Tp = T + DMAX
    d2 = delays.astype(jnp.int32).reshape(1, C)
    grid = (B, C // CB)
    return pl.pallas_call(
        _shift_kernel,
        grid=grid,
        in_specs=[
            pl.BlockSpec((1, CB), lambda b, c: (0, c)),
            pl.BlockSpec((1, T, CB), lambda b, c: (b, 0, c)),
        ],
        out_specs=pl.BlockSpec((1, Tp, CB), lambda b, c: (b, 0, c)),
        out_shape=jax.ShapeDtypeStruct((B, Tp, C), x.dtype),
        compiler_params=pltpu.CompilerParams(
            dimension_semantics=("arbitrary", "arbitrary"),
        ),
    )(d2, x)
